# baseline (speedup 1.0000x reference)
import functools

import jax
import jax.numpy as jnp
from jax import lax
from jax.experimental import pallas as pl
from jax.experimental.pallas import tpu as pltpu
from jax.experimental.pallas import tpu_sc as plsc

_B = 16384
_PAD = 50
_D = 64
_V = 60
_NC, _NS = 2, 16
_NW = _NC * _NS
_PER_W = _B // _NW
_GB = 8
_NGROUP = _PER_W // _GB
_T = _NGROUP // 2


def _make_emb():
    mesh = plsc.VectorSubcoreMesh(core_axis_name="c", subcore_axis_name="s")

    @functools.partial(
        pl.kernel,
        mesh=mesh,
        out_type=jax.ShapeDtypeStruct((_B, _PAD, _D), jnp.float32),
        scratch_types=[
            pltpu.VMEM_SHARED((_V, _D), jnp.float32),
            pltpu.VMEM((2, _GB, _PAD), jnp.int32),
            pltpu.VMEM((2, _GB, _PAD, _D), jnp.float32),
            pltpu.SemaphoreType.DMA,
            pltpu.SemaphoreType.DMA,
            pltpu.SemaphoreType.DMA,
            pltpu.SemaphoreType.DMA,
            pltpu.SemaphoreType.DMA,
            pltpu.SemaphoreType.DMA,
        ],
        compiler_params=pltpu.CompilerParams(use_tc_tiling_on_sc=False),
    )
    def emb(idx_hbm, table_hbm, out_hbm, table_v, idx_v, rows_v,
            isem0, isem1, gsem0, gsem1, osem0, osem1):
        wid = lax.axis_index("s") * _NC + lax.axis_index("c")
        base = wid * _PER_W
        isems = (isem0, isem1)
        gsems = (gsem0, gsem1)
        osems = (osem0, osem1)

        def idx_cp(g, b):
            return pltpu.make_async_copy(
                idx_hbm.at[pl.ds(base + g * _GB, _GB)], idx_v.at[b], isems[b])

        def gather_cp(b, j):
            return pltpu.make_async_copy(
                table_v.at[idx_v.at[b, j]], rows_v.at[b, j], gsems[b])

        def out_cp(g, b):
            return pltpu.make_async_copy(
                rows_v.at[b], out_hbm.at[pl.ds(base + g * _GB, _GB)], osems[b])

        @pl.when(lax.axis_index("s") == 0)
        def _():
            pltpu.sync_copy(table_hbm, table_v)
        plsc.subcore_barrier()

        idx_cp(0, 0).start()
        idx_cp(1, 1).start()

        def body(t, carry):
            for b in range(2):
                g = 2 * t + b
                idx_cp(g, b).wait()

                @pl.when(t >= 1)
                def _():
                    out_cp(g - 2, b).wait()

                for j in range(_GB):
                    gather_cp(b, j).start()
                for j in range(_GB):
                    gather_cp(b, j).wait()

                out_cp(g, b).start()

                @pl.when(t < _T - 1)
                def _():
                    idx_cp(g + 2, b).start()
            return carry

        lax.fori_loop(0, _T, body, 0)

        out_cp(_NGROUP - 2, 0).wait()
        out_cp(_NGROUP - 1, 1).wait()

    return emb


_emb = _make_emb()


@jax.jit
def kernel(indices, emb_weight):
    return _emb(indices, emb_weight)


# speedup vs baseline: 2.4419x; 2.4419x over previous
import functools

import jax
import jax.numpy as jnp
from jax import lax
from jax.experimental import pallas as pl
from jax.experimental.pallas import tpu as pltpu
from jax.experimental.pallas import tpu_sc as plsc

_B = 16384
_PAD = 50
_D = 64
_V = 60
_BB = 2048
_NB = _B // _BB


def _tc_body(idx_ref, tab_ref, out_ref):
    idx = idx_ref[0, 0, :]
    oh = (lax.broadcasted_iota(jnp.int32, (_D, _BB), 0)
          == idx[None, :]).astype(jnp.float32)
    out_ref[0] = jnp.dot(tab_ref[...], oh,
                         preferred_element_type=jnp.float32)


_tc_emb = pl.pallas_call(
    _tc_body,
    grid=(_PAD, _NB),
    in_specs=[
        pl.BlockSpec((1, 1, _BB), lambda p, ib: (p, 0, ib)),
        pl.BlockSpec((_D, _D), lambda p, ib: (0, 0)),
    ],
    out_specs=pl.BlockSpec((1, _D, _BB), lambda p, ib: (p, 0, ib)),
    out_shape=jax.ShapeDtypeStruct((_PAD, _D, _B), jnp.float32),
)


@jax.jit
def kernel(indices, emb_weight):
    idx_t = indices.T.reshape(_PAD, 1, _B)
    tab_t = jnp.pad(emb_weight, ((0, _D - _V), (0, 0))).T
    out_t = _tc_emb(idx_t, tab_t)
    return out_t.transpose(2, 0, 1)
